# baseline (speedup 1.0000x reference)
import functools

import jax
import jax.numpy as jnp
from jax import lax
from jax.experimental import pallas as pl
from jax.experimental.pallas import tpu as pltpu
from jax.experimental.pallas import tpu_sc as plsc

N = 100000
D = 128
B = 1024
OUT = 128
BLK = 512
CHUNK = 256
N_PAD = ((N + BLK - 1) // BLK) * BLK
NBLK = N_PAD // BLK
NEG_INF = float("-inf")

NTILE = 32
PER_TILE = N_PAD // NTILE
SC_CH = 112
NCH = PER_TILE // SC_CH
IMG_ROWS = B + 8


def _sc_sum_body(feats_hbm, ids_hbm, w_hbm, img_out, img,
                 feats_v, wf_v, ids_v, w_v):
    cid = lax.axis_index("c")
    sid = lax.axis_index("s")

    def _zrow(i, _):
        for j in range(D // 16):
            wf_v[i, pl.ds(16 * j, 16)] = jnp.zeros((16,), jnp.float32)
        return 0

    lax.fori_loop(0, SC_CH, _zrow, 0)
    rows_per_tile = B // 16
    pltpu.sync_copy(wf_v.at[pl.ds(0, rows_per_tile)],
                    img.at[pl.ds(sid * rows_per_tile, rows_per_tile)])

    @pl.when(sid == 0)
    def _zdump():
        pltpu.sync_copy(wf_v.at[pl.ds(0, 8)], img.at[pl.ds(B, 8)])

    plsc.subcore_barrier()
    tile_base = (cid * 16 + sid) * PER_TILE

    gdn = lax.GatherDimensionNumbers(
        offset_dims=(), collapsed_slice_dims=(0,), start_index_map=(0,))

    def _chunk(ch, _):
        base = tile_base + ch * SC_CH
        pltpu.sync_copy(feats_hbm.at[pl.ds(base, SC_CH)], feats_v)
        pltpu.sync_copy(ids_hbm.at[pl.ds(base, SC_CH)], ids_v)
        pltpu.sync_copy(w_hbm.at[pl.ds(base, SC_CH)], w_v)

        def _group(gi, _):
            wvec = w_v[pl.ds(gi * 16, 16)]
            for k in range(16):
                g = lax.gather(
                    wvec, jnp.full((16, 1), k, jnp.int32), gdn, (1,),
                    mode=lax.GatherScatterMode.PROMISE_IN_BOUNDS)
                n = gi * 16 + k
                for j in range(D // 16):
                    sl = pl.ds(16 * j, 16)
                    wf_v[n, sl] = feats_v[n, sl] * g
            return 0

        lax.fori_loop(0, SC_CH // 16, _group, 0)
        pltpu.sync_copy(wf_v, img.at[ids_v], add=True)
        return 0

    lax.fori_loop(0, NCH, _chunk, 0)

    plsc.subcore_barrier()
    pltpu.sync_copy(img.at[pl.ds(sid * rows_per_tile, rows_per_tile)],
                    img_out.at[cid, pl.ds(sid * rows_per_tile,
                                          rows_per_tile)])


def _sc_sum(feats_p, ids_p, w_flat):
    mesh = plsc.VectorSubcoreMesh(core_axis_name="c", subcore_axis_name="s")
    return pl.kernel(
        _sc_sum_body,
        mesh=mesh,
        out_type=jax.ShapeDtypeStruct((2, B, D), jnp.float32),
        scratch_types=[
            pltpu.VMEM_SHARED((IMG_ROWS, D), jnp.float32),
            pltpu.VMEM((SC_CH, D), jnp.float32),
            pltpu.VMEM((SC_CH, D), jnp.float32),
            pltpu.VMEM((SC_CH,), jnp.int32),
            pltpu.VMEM((SC_CH,), jnp.float32),
        ],
    )(feats_p, ids_p, w_flat)


def _gate_body(feats_ref, waw_ref, baw_ref, w_ref):
    gate_logit = jnp.sum(feats_ref[...] * waw_ref[...], axis=1,
                         keepdims=True)
    w_ref[...] = jax.nn.sigmoid(gate_logit + baw_ref[0, 0])


def _tc_gate(feats_p, waw_row, baw):
    return pl.pallas_call(
        _gate_body,
        grid=(NBLK,),
        in_specs=[
            pl.BlockSpec((BLK, D), lambda i: (i, 0)),
            pl.BlockSpec((1, D), lambda i: (0, 0)),
            pl.BlockSpec((1, 1), lambda i: (0, 0)),
        ],
        out_specs=pl.BlockSpec((BLK, 1), lambda i: (i, 0)),
        out_shape=jax.ShapeDtypeStruct((N_PAD, 1), jnp.float32),
    )(feats_p, waw_row, baw)


def _max_body(info_ref, ids_ref, idc_ref, feats_ref, max_ref):
    i = pl.program_id(0)

    @pl.when(i == 0)
    def _init():
        max_ref[...] = jnp.full((B, D), NEG_INF, jnp.float32)

    feats = feats_ref[...]
    ids = ids_ref[0, 0, :]
    ids_col = idc_ref[0, :, :]

    mx = feats
    s = 1
    while s < BLK:
        mx_sh = jnp.concatenate(
            [jnp.full((s, D), NEG_INF, jnp.float32), mx[:-s, :]], axis=0)
        ids_sh = jnp.concatenate(
            [jnp.full((s, 1), -1, jnp.int32), ids_col[:-s, :]], axis=0)
        mx = jnp.where(ids_col == ids_sh, jnp.maximum(mx, mx_sh), mx)
        s *= 2

    nxt_col = jnp.concatenate(
        [ids_col[1:, :], jnp.full((1, 1), -1, jnp.int32)], axis=0)
    run_end_col = ids_col != nxt_col
    nxt = jnp.concatenate([ids[1:], jnp.full((1,), -1, jnp.int32)])
    ids_re = jnp.where(ids != nxt, ids, -1)
    mx_re = jnp.where(run_end_col, mx, 0.0)

    lo = info_ref[i, 0]
    hi = info_ref[i, 1]
    for c in range(B // CHUNK):

        @pl.when(jnp.logical_and(lo < (c + 1) * CHUNK, hi >= c * CHUNK))
        def _scatter(c=c):
            seg = c * CHUNK + jax.lax.broadcasted_iota(
                jnp.int32, (CHUNK, BLK), 0)
            onehot_re = (ids_re[None, :] == seg).astype(jnp.float32)
            sel = jax.lax.dot_general(
                onehot_re, mx_re, (((1,), (0,)), ((), ())),
                preferred_element_type=jnp.float32)
            present = jnp.sum(onehot_re, axis=1, keepdims=True)
            rows = pl.ds(c * CHUNK, CHUNK)
            max_ref[rows, :] = jnp.where(
                present > 0, jnp.maximum(max_ref[rows, :], sel),
                max_ref[rows, :])


def _tc_max(info, ids_lane, ids_sub, feats_p):
    grid_spec = pltpu.PrefetchScalarGridSpec(
        num_scalar_prefetch=1,
        grid=(NBLK,),
        in_specs=[
            pl.BlockSpec((1, 1, BLK), lambda i, info: (i, 0, 0)),
            pl.BlockSpec((1, BLK, 1), lambda i, info: (i, 0, 0)),
            pl.BlockSpec((BLK, D), lambda i, info: (i, 0)),
        ],
        out_specs=pl.BlockSpec((B, D), lambda i, info: (0, 0)),
    )
    return pl.pallas_call(
        _max_body,
        grid_spec=grid_spec,
        out_shape=jax.ShapeDtypeStruct((B, D), jnp.float32),
        compiler_params=pltpu.CompilerParams(
            dimension_semantics=("arbitrary",)),
    )(info, ids_lane, ids_sub, feats_p)


def _fin_body(img_ref, max_ref, wlin_ref, blin_ref, out_ref):
    img = img_ref[...]
    h = jnp.concatenate([img[0] + img[1], max_ref[...]], axis=1)
    out_ref[...] = jax.lax.dot_general(
        h, wlin_ref[...], (((1,), (0,)), ((), ())),
        preferred_element_type=jnp.float32) + blin_ref[...]


def _tc_fin(img, maxp, W_lin, blin):
    return pl.pallas_call(
        _fin_body,
        out_shape=jax.ShapeDtypeStruct((B, OUT), jnp.float32),
    )(img, maxp, W_lin, blin)


@jax.jit
def kernel(feats, segment_ids, W_aw, b_aw, W_lin, b_lin):
    ids = segment_ids.astype(jnp.int32)
    pad = N_PAD - N
    feats_p = jnp.pad(feats, ((0, pad), (0, 0)))
    ids_p = jnp.pad(ids, (0, pad), constant_values=B)
    blk_ids = ids_p.reshape(NBLK, BLK)
    info = jnp.stack([blk_ids[:, 0], blk_ids[:, -1]], axis=1)
    ids_lane = ids_p.reshape(NBLK, 1, BLK)
    ids_sub = ids_p.reshape(NBLK, BLK, 1)
    waw_row = W_aw.reshape(1, D)
    baw = b_aw.reshape(1, 1)
    blin = b_lin.reshape(1, OUT)

    w_flat = _tc_gate(feats_p, waw_row, baw).reshape(N_PAD)
    img = _sc_sum(feats_p, ids_p, w_flat)
    maxp = _tc_max(info, ids_lane, ids_sub, feats_p)
    return _tc_fin(img, maxp, W_lin, blin)


# speedup vs baseline: 1.4668x; 1.4668x over previous
import functools

import jax
import jax.numpy as jnp
from jax import lax
from jax.experimental import pallas as pl
from jax.experimental.pallas import tpu as pltpu
from jax.experimental.pallas import tpu_sc as plsc

N = 100000
D = 128
B = 1024
OUT = 128
BLK = 512
CHUNK = 256
N_PAD = ((N + BLK - 1) // BLK) * BLK
NBLK = N_PAD // BLK
NEG_INF = float("-inf")

NTILE = 32
PER_TILE = N_PAD // NTILE
SC_CH = 112
NCH = PER_TILE // SC_CH
IMG_ROWS = B + 8


def _sc_sum_body(feats_hbm, ids_hbm, waw_hbm, baw_hbm, img_out, img,
                 feats_v, wf_v, ids_v, waw_v, baw_v):
    cid = lax.axis_index("c")
    sid = lax.axis_index("s")

    def _zrow(i, _):
        for j in range(D // 16):
            wf_v[i, pl.ds(16 * j, 16)] = jnp.zeros((16,), jnp.float32)
        return 0

    lax.fori_loop(0, SC_CH, _zrow, 0)
    rows_per_tile = B // 16
    pltpu.sync_copy(wf_v.at[pl.ds(0, rows_per_tile)],
                    img.at[pl.ds(sid * rows_per_tile, rows_per_tile)])

    @pl.when(sid == 0)
    def _zdump():
        pltpu.sync_copy(wf_v.at[pl.ds(0, 8)], img.at[pl.ds(B, 8)])

    pltpu.sync_copy(waw_hbm, waw_v)
    pltpu.sync_copy(baw_hbm, baw_v)
    plsc.subcore_barrier()

    waw = [waw_v[pl.ds(16 * j, 16)] for j in range(D // 16)]
    baw = baw_v[...]
    tile_base = (cid * 16 + sid) * PER_TILE

    for ch in range(NCH):
        base = tile_base + ch * SC_CH
        pltpu.sync_copy(feats_hbm.at[pl.ds(base, SC_CH)], feats_v)
        pltpu.sync_copy(ids_hbm.at[pl.ds(base, SC_CH)], ids_v)

        def _node(n, _):
            row = [feats_v[n, pl.ds(16 * j, 16)] for j in range(D // 16)]
            acc = row[0] * waw[0]
            for j in range(1, D // 16):
                acc = acc + row[j] * waw[j]
            lane = lax.iota(jnp.int32, 16)
            gdn = lax.GatherDimensionNumbers(
                offset_dims=(), collapsed_slice_dims=(0,),
                start_index_map=(0,))
            for s in (8, 4, 2, 1):
                acc = acc + lax.gather(
                    acc, (lane ^ s)[:, None], gdn, (1,),
                    mode=lax.GatherScatterMode.PROMISE_IN_BOUNDS)
            sv = acc + baw
            g = 1.0 / (1.0 + jnp.exp(-sv))
            for j in range(D // 16):
                wf_v[n, pl.ds(16 * j, 16)] = row[j] * g
            return 0

        lax.fori_loop(0, SC_CH, _node, 0)
        pltpu.sync_copy(wf_v, img.at[ids_v], add=True)

    plsc.subcore_barrier()
    pltpu.sync_copy(img.at[pl.ds(sid * rows_per_tile, rows_per_tile)],
                    img_out.at[cid, pl.ds(sid * rows_per_tile,
                                          rows_per_tile)])


def _sc_sum(feats_p, ids_p, waw_flat, baw16):
    mesh = plsc.VectorSubcoreMesh(core_axis_name="c", subcore_axis_name="s")
    return pl.kernel(
        _sc_sum_body,
        mesh=mesh,
        out_type=jax.ShapeDtypeStruct((2, B, D), jnp.float32),
        scratch_types=[
            pltpu.VMEM_SHARED((IMG_ROWS, D), jnp.float32),
            pltpu.VMEM((SC_CH, D), jnp.float32),
            pltpu.VMEM((SC_CH, D), jnp.float32),
            pltpu.VMEM((SC_CH,), jnp.int32),
            pltpu.VMEM((D,), jnp.float32),
            pltpu.VMEM((16,), jnp.float32),
        ],
    )(feats_p, ids_p, waw_flat, baw16)


def _max_body(info_ref, ids_ref, idc_ref, feats_ref, max_ref):
    i = pl.program_id(0)

    @pl.when(i == 0)
    def _init():
        max_ref[...] = jnp.full((B, D), NEG_INF, jnp.float32)

    feats = feats_ref[...]
    ids = ids_ref[0, 0, :]
    ids_col = idc_ref[0, :, :]

    mx = feats
    s = 1
    while s < BLK:
        mx_sh = jnp.concatenate(
            [jnp.full((s, D), NEG_INF, jnp.float32), mx[:-s, :]], axis=0)
        ids_sh = jnp.concatenate(
            [jnp.full((s, 1), -1, jnp.int32), ids_col[:-s, :]], axis=0)
        mx = jnp.where(ids_col == ids_sh, jnp.maximum(mx, mx_sh), mx)
        s *= 2

    nxt_col = jnp.concatenate(
        [ids_col[1:, :], jnp.full((1, 1), -1, jnp.int32)], axis=0)
    run_end_col = ids_col != nxt_col
    nxt = jnp.concatenate([ids[1:], jnp.full((1,), -1, jnp.int32)])
    ids_re = jnp.where(ids != nxt, ids, -1)
    mx_re = jnp.where(run_end_col, mx, 0.0)

    lo = info_ref[i, 0]
    hi = info_ref[i, 1]
    for c in range(B // CHUNK):

        @pl.when(jnp.logical_and(lo < (c + 1) * CHUNK, hi >= c * CHUNK))
        def _scatter(c=c):
            seg = c * CHUNK + jax.lax.broadcasted_iota(
                jnp.int32, (CHUNK, BLK), 0)
            onehot_re = (ids_re[None, :] == seg).astype(jnp.float32)
            sel = jax.lax.dot_general(
                onehot_re, mx_re, (((1,), (0,)), ((), ())),
                preferred_element_type=jnp.float32)
            present = jnp.sum(onehot_re, axis=1, keepdims=True)
            rows = pl.ds(c * CHUNK, CHUNK)
            max_ref[rows, :] = jnp.where(
                present > 0, jnp.maximum(max_ref[rows, :], sel),
                max_ref[rows, :])


def _tc_max(info, ids_lane, ids_sub, feats_p):
    grid_spec = pltpu.PrefetchScalarGridSpec(
        num_scalar_prefetch=1,
        grid=(NBLK,),
        in_specs=[
            pl.BlockSpec((1, 1, BLK), lambda i, info: (i, 0, 0)),
            pl.BlockSpec((1, BLK, 1), lambda i, info: (i, 0, 0)),
            pl.BlockSpec((BLK, D), lambda i, info: (i, 0)),
        ],
        out_specs=pl.BlockSpec((B, D), lambda i, info: (0, 0)),
    )
    return pl.pallas_call(
        _max_body,
        grid_spec=grid_spec,
        out_shape=jax.ShapeDtypeStruct((B, D), jnp.float32),
        compiler_params=pltpu.CompilerParams(
            dimension_semantics=("arbitrary",)),
    )(info, ids_lane, ids_sub, feats_p)


def _fin_body(img_ref, max_ref, wlin_ref, blin_ref, out_ref):
    img = img_ref[...]
    h = jnp.concatenate([img[0] + img[1], max_ref[...]], axis=1)
    out_ref[...] = jax.lax.dot_general(
        h, wlin_ref[...], (((1,), (0,)), ((), ())),
        preferred_element_type=jnp.float32) + blin_ref[...]


def _tc_fin(img, maxp, W_lin, blin):
    return pl.pallas_call(
        _fin_body,
        out_shape=jax.ShapeDtypeStruct((B, OUT), jnp.float32),
    )(img, maxp, W_lin, blin)


@jax.jit
def kernel(feats, segment_ids, W_aw, b_aw, W_lin, b_lin):
    ids = segment_ids.astype(jnp.int32)
    pad = N_PAD - N
    feats_p = jnp.pad(feats, ((0, pad), (0, 0)))
    ids_p = jnp.pad(ids, (0, pad), constant_values=B)
    blk_ids = ids_p.reshape(NBLK, BLK)
    info = jnp.stack([blk_ids[:, 0], blk_ids[:, -1]], axis=1)
    ids_lane = ids_p.reshape(NBLK, 1, BLK)
    ids_sub = ids_p.reshape(NBLK, BLK, 1)
    waw_flat = W_aw.reshape(D)
    baw16 = jnp.full((16,), b_aw[0], jnp.float32)
    blin = b_lin.reshape(1, OUT)

    img = _sc_sum(feats_p, ids_p, waw_flat, baw16)
    maxp = _tc_max(info, ids_lane, ids_sub, feats_p)
    return _tc_fin(img, maxp, W_lin, blin)
